# Initial kernel scaffold; baseline (speedup 1.0000x reference)
import jax
import jax.numpy as jnp
from jax.experimental import pallas as pl


def kernel(instances):
    raise NotImplementedError("write your pallas kernel here")


# speedup vs baseline: 1.6004x; 1.6004x over previous
import functools

import jax
import jax.numpy as jnp
import numpy as np
from jax import lax
from jax.experimental import pallas as pl
from jax.experimental.pallas import tpu as pltpu
from jax.experimental.pallas import tpu_sc as plsc

DROP_RATE = 0.15
NUM_ROWS = 16384
ROW_DIM = 64
NUM_KEEP = max(1, int(NUM_ROWS * (1.0 - DROP_RATE)))

NUM_CORES = 2
NUM_SUBCORES = 16
NUM_WORKERS = NUM_CORES * NUM_SUBCORES

ROWS_PER_WORKER = -(-NUM_KEEP // NUM_WORKERS)
CHUNK = 128
NUM_CHUNKS = -(-ROWS_PER_WORKER // CHUNK)
SLOTS = NUM_CHUNKS * CHUNK


@functools.lru_cache(maxsize=1)
def _index_arrays():
    with jax.ensure_compile_time_eval():
        perm = jax.random.permutation(jax.random.key(42), NUM_ROWS)
    idx = np.asarray(perm)[:NUM_KEEP].astype(np.int32)
    pos = np.minimum(
        np.arange(NUM_WORKERS)[:, None] * ROWS_PER_WORKER + np.arange(SLOTS)[None, :],
        NUM_KEEP - 1,
    ).astype(np.int32)
    return idx[pos], pos.reshape(NUM_WORKERS, NUM_CHUNKS, CHUNK)


@functools.lru_cache(maxsize=1)
def _build_gather():
    mesh = plsc.VectorSubcoreMesh(core_axis_name="c", subcore_axis_name="s")

    @functools.partial(
        pl.kernel,
        out_type=jax.ShapeDtypeStruct((NUM_KEEP, ROW_DIM), jnp.float32),
        mesh=mesh,
        compiler_params=pltpu.CompilerParams(use_tc_tiling_on_sc=False),
        scratch_types=[
            pltpu.VMEM((SLOTS,), jnp.int32),
            pltpu.VMEM((NUM_CHUNKS, CHUNK), jnp.int32),
            pltpu.VMEM((SLOTS, ROW_DIM), jnp.float32),
            pltpu.SemaphoreType.DMA,
        ],
    )
    def _gather_rows(table_hbm, sidx_hbm, didx_hbm, out_hbm, sidx_v, didx_v, rows_v, sem):
        wid = lax.axis_index("s") * NUM_CORES + lax.axis_index("c")
        pltpu.sync_copy(sidx_hbm.at[wid], sidx_v)
        pltpu.sync_copy(didx_hbm.at[wid], didx_v)
        gathers = [
            pltpu.async_copy(
                table_hbm.at[sidx_v.at[pl.ds(c * CHUNK, CHUNK)]],
                rows_v.at[pl.ds(c * CHUNK, CHUNK)],
                sem,
            )
            for c in range(NUM_CHUNKS)
        ]
        for g in gathers:
            g.wait()
        scatters = [
            pltpu.async_copy(
                rows_v.at[pl.ds(c * CHUNK, CHUNK)],
                out_hbm.at[didx_v.at[c]],
                sem,
            )
            for c in range(NUM_CHUNKS)
        ]
        for s in scatters:
            s.wait()

    return _gather_rows


def kernel(instances):
    src_idx, dst_idx = _index_arrays()
    return _build_gather()(instances, jnp.asarray(src_idx), jnp.asarray(dst_idx))


# speedup vs baseline: 1.7920x; 1.1197x over previous
import functools

import jax
import jax.numpy as jnp
import numpy as np
from jax import lax
from jax.experimental import pallas as pl
from jax.experimental.pallas import tpu as pltpu
from jax.experimental.pallas import tpu_sc as plsc

DROP_RATE = 0.15
NUM_ROWS = 16384
ROW_DIM = 64
NUM_KEEP = max(1, int(NUM_ROWS * (1.0 - DROP_RATE)))

NUM_CORES = 2
NUM_SUBCORES = 16
NUM_WORKERS = NUM_CORES * NUM_SUBCORES

ROWS_PER_WORKER = -(-NUM_KEEP // NUM_WORKERS)
LAST_BASE = NUM_KEEP - ROWS_PER_WORKER
CHUNKS = [(0, 112), (112, 112), (224, 112), (336, 100)]
assert sum(sz for _, sz in CHUNKS) == ROWS_PER_WORKER


@functools.lru_cache(maxsize=1)
def _worker_indices() -> np.ndarray:
    with jax.ensure_compile_time_eval():
        perm = jax.random.permutation(jax.random.key(42), NUM_ROWS)
    idx = np.asarray(perm)[:NUM_KEEP].astype(np.int32)
    bases = np.minimum(np.arange(NUM_WORKERS) * ROWS_PER_WORKER, LAST_BASE)
    return np.stack([idx[b : b + ROWS_PER_WORKER] for b in bases])


@functools.lru_cache(maxsize=1)
def _build_gather():
    mesh = plsc.VectorSubcoreMesh(core_axis_name="c", subcore_axis_name="s")

    @functools.partial(
        pl.kernel,
        out_type=jax.ShapeDtypeStruct((NUM_KEEP, ROW_DIM), jnp.float32),
        mesh=mesh,
        compiler_params=pltpu.CompilerParams(use_tc_tiling_on_sc=False),
        scratch_types=[
            pltpu.VMEM((ROWS_PER_WORKER,), jnp.int32),
            pltpu.VMEM((ROWS_PER_WORKER, ROW_DIM), jnp.float32),
            pltpu.SemaphoreType.DMA,
        ],
    )
    def _gather_rows(table_hbm, sidx_hbm, out_hbm, sidx_v, rows_v, sem):
        wid = lax.axis_index("s") * NUM_CORES + lax.axis_index("c")
        base = jnp.minimum(wid * ROWS_PER_WORKER, LAST_BASE)
        pltpu.sync_copy(sidx_hbm.at[wid], sidx_v)
        gathers = [
            pltpu.async_copy(
                table_hbm.at[sidx_v.at[pl.ds(off, sz)]],
                rows_v.at[pl.ds(off, sz)],
                sem,
            )
            for off, sz in CHUNKS
        ]
        for g in gathers:
            g.wait()
        pltpu.sync_copy(rows_v, out_hbm.at[pl.ds(base, ROWS_PER_WORKER)])

    return _gather_rows


def kernel(instances):
    return _build_gather()(instances, jnp.asarray(_worker_indices()))


# speedup vs baseline: 1.7955x; 1.0020x over previous
import functools

import jax
import jax.numpy as jnp
import numpy as np
from jax import lax
from jax.experimental import pallas as pl
from jax.experimental.pallas import tpu as pltpu
from jax.experimental.pallas import tpu_sc as plsc

DROP_RATE = 0.15
NUM_ROWS = 16384
ROW_DIM = 64
NUM_KEEP = max(1, int(NUM_ROWS * (1.0 - DROP_RATE)))

NUM_CORES = 2
NUM_SUBCORES = 16
NUM_WORKERS = NUM_CORES * NUM_SUBCORES

ROWS_PER_WORKER = -(-NUM_KEEP // NUM_WORKERS)
LAST_BASE = NUM_KEEP - ROWS_PER_WORKER
IDX_STRIDE = 440
CHUNKS = [(0, 112), (112, 112), (224, 112), (336, 100)]
assert sum(sz for _, sz in CHUNKS) == ROWS_PER_WORKER


@functools.lru_cache(maxsize=1)
def _worker_indices() -> np.ndarray:
    with jax.ensure_compile_time_eval():
        perm = jax.random.permutation(jax.random.key(42), NUM_ROWS)
    idx = np.asarray(perm)[:NUM_KEEP].astype(np.int32)
    bases = np.minimum(np.arange(NUM_WORKERS) * ROWS_PER_WORKER, LAST_BASE)
    flat = np.zeros(NUM_WORKERS * IDX_STRIDE, dtype=np.int32)
    for w, b in enumerate(bases):
        flat[w * IDX_STRIDE : w * IDX_STRIDE + ROWS_PER_WORKER] = idx[b : b + ROWS_PER_WORKER]
    return flat


@functools.lru_cache(maxsize=1)
def _build_gather():
    mesh = plsc.VectorSubcoreMesh(core_axis_name="c", subcore_axis_name="s")

    @functools.partial(
        pl.kernel,
        out_type=jax.ShapeDtypeStruct((NUM_KEEP, ROW_DIM), jnp.float32),
        mesh=mesh,
        compiler_params=pltpu.CompilerParams(use_tc_tiling_on_sc=False),
        scratch_types=[
            pltpu.VMEM((ROWS_PER_WORKER,), jnp.int32),
            pltpu.VMEM((ROWS_PER_WORKER, ROW_DIM), jnp.float32),
            pltpu.SemaphoreType.DMA,
            pltpu.SemaphoreType.DMA,
        ],
    )
    def _gather_rows(table_hbm, sidx_hbm, out_hbm, sidx_v, rows_v, gsem, ssem):
        wid = lax.axis_index("s") * NUM_CORES + lax.axis_index("c")
        base = jnp.minimum(wid * ROWS_PER_WORKER, LAST_BASE)
        pltpu.sync_copy(sidx_hbm.at[pl.ds(wid * IDX_STRIDE, ROWS_PER_WORKER)], sidx_v)
        gathers = [
            pltpu.async_copy(
                table_hbm.at[sidx_v.at[pl.ds(off, sz)]],
                rows_v.at[pl.ds(off, sz)],
                gsem,
            )
            for off, sz in CHUNKS
        ]
        stores = []
        for g, (off, sz) in zip(gathers, CHUNKS):
            g.wait()
            stores.append(
                pltpu.async_copy(
                    rows_v.at[pl.ds(off, sz)],
                    out_hbm.at[pl.ds(base + off, sz)],
                    ssem,
                )
            )
        for s in stores:
            s.wait()

    return _gather_rows


def kernel(instances):
    return _build_gather()(instances, jnp.asarray(_worker_indices()))


# speedup vs baseline: 2.4438x; 1.3611x over previous
import functools

import jax
import jax.numpy as jnp
import numpy as np
from jax import lax
from jax.experimental import pallas as pl
from jax.experimental.pallas import tpu as pltpu
from jax.experimental.pallas import tpu_sc as plsc

DROP_RATE = 0.15
NUM_ROWS = 16384
ROW_DIM = 64
NUM_KEEP = max(1, int(NUM_ROWS * (1.0 - DROP_RATE)))

NUM_CORES = 2
NUM_SUBCORES = 16
NUM_WORKERS = NUM_CORES * NUM_SUBCORES
ROWS_PER_WORKER = ROW_DIM // NUM_WORKERS

LANES = 16
NUM_STEPS = -(-NUM_KEEP // LANES)
KEEP_PAD = NUM_STEPS * LANES


@functools.lru_cache(maxsize=1)
def _gather_indices() -> np.ndarray:
    with jax.ensure_compile_time_eval():
        perm = jax.random.permutation(jax.random.key(42), NUM_ROWS)
    idx = np.asarray(perm)[:NUM_KEEP].astype(np.int32)
    return np.concatenate([idx, np.full(KEEP_PAD - NUM_KEEP, idx[-1], np.int32)])


@functools.lru_cache(maxsize=1)
def _build_gather():
    mesh = plsc.VectorSubcoreMesh(core_axis_name="c", subcore_axis_name="s")

    @functools.partial(
        pl.kernel,
        out_type=jax.ShapeDtypeStruct((ROW_DIM, KEEP_PAD), jnp.float32),
        mesh=mesh,
        compiler_params=pltpu.CompilerParams(
            use_tc_tiling_on_sc=False, needs_layout_passes=False
        ),
        scratch_types=[
            pltpu.VMEM((KEEP_PAD,), jnp.int32),
            pltpu.VMEM((ROWS_PER_WORKER, NUM_ROWS), jnp.float32),
            pltpu.VMEM((ROWS_PER_WORKER, KEEP_PAD), jnp.float32),
            pltpu.SemaphoreType.DMA,
        ],
    )
    def _gather_cols(xt_hbm, idx_hbm, out_hbm, idx_v, tbl_v, res_v, sem):
        wid = lax.axis_index("s") * NUM_CORES + lax.axis_index("c")
        row0 = wid * ROWS_PER_WORKER
        cp_idx = pltpu.async_copy(idx_hbm, idx_v, sem)
        cp_tbl = pltpu.async_copy(
            xt_hbm.at[pl.ds(row0, ROWS_PER_WORKER)], tbl_v, sem
        )
        cp_idx.wait()
        cp_tbl.wait()

        def step(p):
            cols = idx_v[pl.ds(p * LANES, LANES)]
            for r in range(ROWS_PER_WORKER):
                rows = jnp.full((LANES,), r, jnp.int32)
                res_v[r, pl.ds(p * LANES, LANES)] = plsc.load_gather(
                    tbl_v, [rows, cols]
                )

        plsc.parallel_loop(0, NUM_STEPS, 1, unroll=8, carry=None)(step)

        for r in range(ROWS_PER_WORKER):
            pltpu.sync_copy(res_v.at[r], out_hbm.at[row0 + r])

    return _gather_cols


def kernel(instances):
    out_t = _build_gather()(instances.T, jnp.asarray(_gather_indices()))
    return out_t[:, :NUM_KEEP].T


# speedup vs baseline: 2.8358x; 1.1604x over previous
import functools

import jax
import jax.numpy as jnp
import numpy as np
from jax import lax
from jax.experimental import pallas as pl
from jax.experimental.pallas import tpu as pltpu
from jax.experimental.pallas import tpu_sc as plsc

DROP_RATE = 0.15
NUM_ROWS = 16384
ROW_DIM = 64
NUM_KEEP = max(1, int(NUM_ROWS * (1.0 - DROP_RATE)))

NUM_CORES = 2
NUM_SUBCORES = 16
NUM_WORKERS = NUM_CORES * NUM_SUBCORES
ROWS_PER_WORKER = ROW_DIM // NUM_WORKERS

LANES = 16
SUBLANES = 8
TILE_MINOR = 128
IN_TILES = NUM_ROWS // TILE_MINOR
OUT_TILES = -(-NUM_KEEP // TILE_MINOR)
KEEP_PAD = OUT_TILES * TILE_MINOR
NUM_STEPS = KEEP_PAD // LANES


@functools.lru_cache(maxsize=1)
def _gather_indices() -> np.ndarray:
    with jax.ensure_compile_time_eval():
        perm = jax.random.permutation(jax.random.key(42), NUM_ROWS)
    idx = np.asarray(perm)[:NUM_KEEP].astype(np.int32)
    return np.concatenate([idx, np.full(KEEP_PAD - NUM_KEEP, idx[-1], np.int32)])


@functools.lru_cache(maxsize=1)
def _build_gather():
    mesh = plsc.VectorSubcoreMesh(core_axis_name="c", subcore_axis_name="s")

    @functools.partial(
        pl.kernel,
        out_type=jax.ShapeDtypeStruct(
            (ROW_DIM // SUBLANES, OUT_TILES, SUBLANES, TILE_MINOR), jnp.float32
        ),
        mesh=mesh,
        compiler_params=pltpu.CompilerParams(
            use_tc_tiling_on_sc=False, needs_layout_passes=False
        ),
        scratch_types=[
            pltpu.VMEM((KEEP_PAD,), jnp.int32),
            pltpu.VMEM((ROWS_PER_WORKER, IN_TILES, TILE_MINOR), jnp.float32),
            pltpu.VMEM((ROWS_PER_WORKER, OUT_TILES, TILE_MINOR), jnp.float32),
            pltpu.SemaphoreType.DMA,
        ],
    )
    def _gather_cols(z_hbm, idx_hbm, y_hbm, idx_v, tbl_v, res_v, sem):
        wid = lax.axis_index("s") * NUM_CORES + lax.axis_index("c")
        copies = [pltpu.async_copy(idx_hbm, idx_v, sem)]
        for r in range(ROWS_PER_WORKER):
            row = wid * ROWS_PER_WORKER + r
            copies.append(
                pltpu.async_copy(
                    z_hbm.at[row // SUBLANES, :, row % SUBLANES, :],
                    tbl_v.at[r],
                    sem,
                )
            )
        for cp in copies:
            cp.wait()

        def step(p):
            cols = idx_v[pl.ds(p * LANES, LANES)]
            b = cols >> 7
            c = cols & (TILE_MINOR - 1)
            q = p // SUBLANES
            off = (p % SUBLANES) * LANES
            for r in range(ROWS_PER_WORKER):
                rows = jnp.full((LANES,), r, jnp.int32)
                res_v[r, q, pl.ds(off, LANES)] = plsc.load_gather(
                    tbl_v, [rows, b, c]
                )

        plsc.parallel_loop(0, NUM_STEPS, 1, unroll=8, carry=None)(step)

        for r in range(ROWS_PER_WORKER):
            row = wid * ROWS_PER_WORKER + r
            pltpu.sync_copy(
                res_v.at[r],
                y_hbm.at[row // SUBLANES, :, row % SUBLANES, :],
            )

    return _gather_cols


def kernel(instances):
    z = instances.T.reshape(
        ROW_DIM // SUBLANES, SUBLANES, IN_TILES, TILE_MINOR
    ).transpose(0, 2, 1, 3)
    y = _build_gather()(z, jnp.asarray(_gather_indices()))
    out_t = y.transpose(0, 2, 1, 3).reshape(ROW_DIM, KEEP_PAD)
    return out_t[:, :NUM_KEEP].T
